# baseline (speedup 1.0000x reference)
import functools

import jax
import jax.numpy as jnp
from jax import lax
from jax.experimental import pallas as pl
from jax.experimental.pallas import tpu as pltpu
from jax.experimental.pallas import tpu_sc as plsc

B, P, C, O = 32, 8732, 21, 20
PPAD = 9216
TROWS = 24
THRESH = 0.5
NW = 32
NU = PPAD // 128
NS_MAX = 3


def _match_body(p4_ref, ttm_ref, bti_ref, msk_ref):
    p4 = p4_ref[...]
    pcx, pcy = p4[0:1, :], p4[1:2, :]
    pw, ph = p4[2:3, :], p4[3:4, :]
    px1 = pcx - pw / 2.0
    py1 = pcy - ph / 2.0
    px2 = pcx + pw / 2.0
    py2 = pcy + ph / 2.0

    ttm = ttm_ref[0]
    tx1 = ttm[:, 0:1]
    ty1 = ttm[:, 1:2]
    tx2 = ttm[:, 2:3]
    ty2 = ttm[:, 3:4]

    iw = jnp.clip(jnp.minimum(tx2, px2) - jnp.maximum(tx1, px1), 0.0, None)
    ih = jnp.clip(jnp.minimum(ty2, py2) - jnp.maximum(ty1, py1), 0.0, None)
    inter = iw * ih
    area_a = (tx2 - tx1) * (ty2 - ty1)
    area_b = (px2 - px1) * (py2 - py1)
    union = area_a + area_b - inter
    ov = inter / union

    rows = lax.broadcasted_iota(jnp.int32, (TROWS, PPAD), 0)
    lanes = lax.broadcasted_iota(jnp.int32, (TROWS, PPAD), 1)

    bto = jnp.max(ov, axis=0, keepdims=True)
    bti = jnp.min(jnp.where(ov == bto, rows, TROWS),
                  axis=0, keepdims=True)
    bpv = jnp.max(ov, axis=1, keepdims=True)
    bpi = jnp.min(jnp.where(ov == bpv, lanes, PPAD),
                  axis=1, keepdims=True)

    match = (lanes == bpi) & (rows < O)
    j_forced = jnp.max(jnp.where(match, rows, -1), axis=0, keepdims=True)
    bti2 = jnp.where(j_forced >= 0, j_forced, bti)
    maskok = (bto >= THRESH) | (j_forced >= 0)
    bti_ref[0] = bti2.astype(jnp.float32)
    msk_ref[0] = maskok.astype(jnp.float32)


def _logf16(x):
    bits = lax.bitcast_convert_type(x, jnp.int32)
    e = ((bits >> 23) & 0xFF) - 126
    m = lax.bitcast_convert_type((bits & 0x007FFFFF) | 0x3F000000,
                                 jnp.float32)
    c = m < 0.7071067811865476
    m = jnp.where(c, m + m, m)
    ef = (e - jnp.where(c, 1, 0)).astype(jnp.float32)
    z = m - 1.0
    zz = z * z
    poly = jnp.full((16,), 7.0376836292e-2, jnp.float32)
    for k in (-1.1514610310e-1, 1.1676998740e-1, -1.2420140846e-1,
              1.4249322787e-1, -1.6668057665e-1, 2.0000714765e-1,
              -2.4999993993e-1, 3.3333331174e-1):
        poly = poly * z + jnp.float32(k)
    y = z * zz * poly
    y = y + ef * jnp.float32(-2.12194440e-4)
    y = y - 0.5 * zz
    return z + y + ef * jnp.float32(0.693359375)


def _sc_encode_body(p4_hbm, t_hbm, bti_hbm, msk_hbm, loc_hbm, conf_hbm,
                    pri_v, rcp_v, t_v, bti_v, msk_v, loc_v, conf_v):
    wid = lax.axis_index("s") * 2 + lax.axis_index("c")
    pltpu.sync_copy(t_hbm, t_v)

    for s in range(9):
        item = wid * 9 + s
        unit = item // 4
        quarter = item - unit * 4
        base = pl.multiple_of(unit * 128, 128)
        ib = pl.multiple_of(quarter * 8, 8)
        pltpu.sync_copy(p4_hbm.at[:, pl.ds(base, 128)], pri_v)
        pltpu.sync_copy(bti_hbm.at[pl.ds(ib, 8), :, pl.ds(base, 128)], bti_v)
        pltpu.sync_copy(msk_hbm.at[pl.ds(ib, 8), :, pl.ds(base, 128)], msk_v)
        for ch in range(8):
            cs = ch * 16
            rcp_v[0, pl.ds(cs, 16)] = 1.0 / (0.1 * pri_v[2, pl.ds(cs, 16)])
            rcp_v[1, pl.ds(cs, 16)] = 1.0 / (0.1 * pri_v[3, pl.ds(cs, 16)])
            rcp_v[2, pl.ds(cs, 16)] = 1.0 / pri_v[2, pl.ds(cs, 16)]
            rcp_v[3, pl.ds(cs, 16)] = 1.0 / pri_v[3, pl.ds(cs, 16)]

        def one_image(i, carry):
            tbase = (ib + i) * 256
            for ch in range(8):
                cs = ch * 16
                idx = bti_v[i, 0, pl.ds(cs, 16)].astype(jnp.int32)
                msk = msk_v[i, 0, pl.ds(cs, 16)]
                ti = tbase + idx
                x1 = plsc.load_gather(t_v, [ti])
                y1 = plsc.load_gather(t_v, [ti + 32])
                x2 = plsc.load_gather(t_v, [ti + 64])
                y2 = plsc.load_gather(t_v, [ti + 96])
                lb = plsc.load_gather(t_v, [ti + 128])
                pcx = pri_v[0, pl.ds(cs, 16)]
                pcy = pri_v[1, pl.ds(cs, 16)]
                g_cx = ((x1 + x2) * 0.5 - pcx) * rcp_v[0, pl.ds(cs, 16)]
                g_cy = ((y1 + y2) * 0.5 - pcy) * rcp_v[1, pl.ds(cs, 16)]
                g_w = _logf16((x2 - x1) * rcp_v[2, pl.ds(cs, 16)]) * 5.0
                g_h = _logf16((y2 - y1) * rcp_v[3, pl.ds(cs, 16)]) * 5.0
                conf = jnp.where(msk != 0.0, (lb + 1.0).astype(jnp.int32), 0)
                loc_v[i, 0, pl.ds(cs, 16)] = g_cx
                loc_v[i, 1, pl.ds(cs, 16)] = g_cy
                loc_v[i, 2, pl.ds(cs, 16)] = g_w
                loc_v[i, 3, pl.ds(cs, 16)] = g_h
                conf_v[i, pl.ds(cs, 16)] = conf
            return carry

        lax.fori_loop(0, 8, one_image, 0)
        pltpu.sync_copy(loc_v, loc_hbm.at[pl.ds(ib, 8), :, pl.ds(base, 128)])
        pltpu.sync_copy(conf_v, conf_hbm.at[pl.ds(ib, 8), pl.ds(base, 128)])


_SC_ENCODE_CACHE = []


def _sc_encode(*args):
    if not _SC_ENCODE_CACHE:
        _SC_ENCODE_CACHE.append(_make_sc_encode())
    return _SC_ENCODE_CACHE[0](*args)


def _make_sc_encode():
    return functools.partial(
        pl.kernel,
        out_type=[
            jax.ShapeDtypeStruct((B, 4, PPAD), jnp.float32),
            jax.ShapeDtypeStruct((B, PPAD), jnp.int32),
        ],
        mesh=plsc.VectorSubcoreMesh(core_axis_name="c", subcore_axis_name="s",
                                    num_cores=2, num_subcores=16),
        compiler_params=pltpu.CompilerParams(needs_layout_passes=False),
        scratch_types=[
            pltpu.VMEM((4, 128), jnp.float32),
            pltpu.VMEM((4, 128), jnp.float32),
            pltpu.VMEM((B * 8 * 32,), jnp.float32),
            pltpu.VMEM((8, 1, 128), jnp.float32),
            pltpu.VMEM((8, 1, 128), jnp.float32),
            pltpu.VMEM((8, 4, 128), jnp.float32),
            pltpu.VMEM((8, 128), jnp.int32),
        ],
    )(_sc_encode_body)


def kernel(loc_data, conf_data, priors, targets):
    del loc_data, conf_data
    pri = priors[:P, :]
    pad_pri = jnp.broadcast_to(jnp.array([-50.0, -50.0, 1.0, 1.0],
                                         jnp.float32), (PPAD - P, 4))
    p4 = jnp.concatenate([pri, pad_pri], axis=0).T

    pad_box = jnp.array([-9.0, -9.0, -8.0, -8.0, 0.0], jnp.float32)
    tgt24 = jnp.concatenate(
        [targets, jnp.broadcast_to(pad_box, (B, TROWS - O, 5))], axis=1)
    ttm = jnp.pad(tgt24, ((0, 0), (0, 0), (0, 128 - 5)))
    tgt32 = jnp.concatenate(
        [targets, jnp.broadcast_to(pad_box, (B, 32 - O, 5))], axis=1)
    t_sc = jnp.pad(jnp.swapaxes(tgt32, 1, 2),
                   ((0, 0), (0, 3), (0, 0))).reshape(-1)

    bti, msk = pl.pallas_call(
        _match_body,
        grid=(B,),
        in_specs=[
            pl.BlockSpec((4, PPAD), lambda i: (0, 0)),
            pl.BlockSpec((1, TROWS, 128), lambda i: (i, 0, 0)),
        ],
        out_specs=[
            pl.BlockSpec((1, 1, PPAD), lambda i: (i, 0, 0)),
            pl.BlockSpec((1, 1, PPAD), lambda i: (i, 0, 0)),
        ],
        out_shape=[
            jax.ShapeDtypeStruct((B, 1, PPAD), jnp.float32),
            jax.ShapeDtypeStruct((B, 1, PPAD), jnp.float32),
        ],
    )(p4, ttm)

    loc_p, conf_p = _sc_encode(p4, t_sc, bti, msk)
    return jnp.swapaxes(loc_p, 1, 2)[:, :P, :], conf_p[:, :P]


# speedup vs baseline: 1.2283x; 1.2283x over previous
import functools

import jax
import jax.numpy as jnp
from jax import lax
from jax.experimental import pallas as pl
from jax.experimental.pallas import tpu as pltpu
from jax.experimental.pallas import tpu_sc as plsc

B, P, C, O = 32, 8732, 21, 20
PPAD = 9216
TROWS = 24
THRESH = 0.5
NW = 32
NU = PPAD // 128
NS_MAX = 3


def _match_body(p4_ref, ttm_ref, bti_ref):
    p4 = p4_ref[...]
    pcx, pcy = p4[0:1, :], p4[1:2, :]
    pw, ph = p4[2:3, :], p4[3:4, :]
    px1 = pcx - pw / 2.0
    py1 = pcy - ph / 2.0
    px2 = pcx + pw / 2.0
    py2 = pcy + ph / 2.0

    ttm = ttm_ref[0]
    tx1 = ttm[:, 0:1]
    ty1 = ttm[:, 1:2]
    tx2 = ttm[:, 2:3]
    ty2 = ttm[:, 3:4]

    iw = jnp.clip(jnp.minimum(tx2, px2) - jnp.maximum(tx1, px1), 0.0, None)
    ih = jnp.clip(jnp.minimum(ty2, py2) - jnp.maximum(ty1, py1), 0.0, None)
    inter = iw * ih
    area_a = (tx2 - tx1) * (ty2 - ty1)
    area_b = (px2 - px1) * (py2 - py1)
    union = area_a + area_b - inter
    ov = inter / union

    rows = lax.broadcasted_iota(jnp.int32, (TROWS, PPAD), 0)
    lanes = lax.broadcasted_iota(jnp.int32, (TROWS, PPAD), 1)

    bto = jnp.max(ov, axis=0, keepdims=True)
    bti = jnp.min(jnp.where(ov == bto, rows, TROWS),
                  axis=0, keepdims=True)
    bpv = jnp.max(ov, axis=1, keepdims=True)
    bpi = jnp.min(jnp.where(ov == bpv, lanes, PPAD),
                  axis=1, keepdims=True)

    match = (lanes == bpi) & (rows < O)
    j_forced = jnp.max(jnp.where(match, rows, -1), axis=0, keepdims=True)
    bti2 = jnp.where(j_forced >= 0, j_forced, bti)
    maskok = (bto >= THRESH) | (j_forced >= 0)
    bti_ref[0] = (bti2 + 32 * maskok.astype(jnp.int32)).astype(jnp.float32)


def _logf16(x):
    bits = lax.bitcast_convert_type(x, jnp.int32)
    e = ((bits >> 23) & 0xFF) - 126
    m = lax.bitcast_convert_type((bits & 0x007FFFFF) | 0x3F000000,
                                 jnp.float32)
    c = m < 0.7071067811865476
    m = jnp.where(c, m + m, m)
    ef = (e - jnp.where(c, 1, 0)).astype(jnp.float32)
    z = m - 1.0
    zz = z * z
    poly = jnp.full((16,), -1.0 / 6.0, jnp.float32)
    for k in (0.2, -0.25, 1.0 / 3.0):
        poly = poly * z + jnp.float32(k)
    y = z * zz * poly
    y = y + ef * jnp.float32(-2.12194440e-4)
    y = y - 0.5 * zz
    return z + y + ef * jnp.float32(0.693359375)


def _sc_encode_body(p4_hbm, t_hbm, enc_hbm, loc_hbm, conf_hbm,
                    pri_v, rcp_v, t_v, enc_v, loc_v, conf_v):
    wid = lax.axis_index("s") * 2 + lax.axis_index("c")
    pltpu.sync_copy(t_hbm, t_v)
    lane = lax.broadcasted_iota(jnp.int32, (16,), 0)

    def process(unit, ib, n):
        base = pl.multiple_of(unit * 128, 128)
        ibm = pl.multiple_of(ib, 8)
        pltpu.sync_copy(p4_hbm.at[:, pl.ds(base, 128)], pri_v)
        pltpu.sync_copy(enc_hbm.at[pl.ds(ibm, n), :, pl.ds(base, 128)],
                        enc_v.at[pl.ds(0, n)])
        for ch in range(8):
            cs = ch * 16
            rcp_v[0, pl.ds(cs, 16)] = 1.0 / (0.1 * pri_v[2, pl.ds(cs, 16)])
            rcp_v[1, pl.ds(cs, 16)] = 1.0 / (0.1 * pri_v[3, pl.ds(cs, 16)])
            rcp_v[2, pl.ds(cs, 16)] = 1.0 / pri_v[2, pl.ds(cs, 16)]
            rcp_v[3, pl.ds(cs, 16)] = 1.0 / pri_v[3, pl.ds(cs, 16)]

        def one_image(i, carry):
            iv = jnp.full((16,), i, jnp.int32)
            tbase = (ibm + i) * 256
            for ch in range(8):
                cs = ch * 16
                col = lane + cs
                ev = enc_v[i, 0, pl.ds(cs, 16)].astype(jnp.int32)
                idx = ev & 31
                ti = tbase + idx
                x1 = plsc.load_gather(t_v, [ti])
                y1 = plsc.load_gather(t_v, [ti + 32])
                x2 = plsc.load_gather(t_v, [ti + 64])
                y2 = plsc.load_gather(t_v, [ti + 96])
                lb = plsc.load_gather(t_v, [ti + 128])
                pcx = pri_v[0, pl.ds(cs, 16)]
                pcy = pri_v[1, pl.ds(cs, 16)]
                g_cx = ((x1 + x2) * 0.5 - pcx) * rcp_v[0, pl.ds(cs, 16)]
                g_cy = ((y1 + y2) * 0.5 - pcy) * rcp_v[1, pl.ds(cs, 16)]
                g_w = _logf16((x2 - x1) * rcp_v[2, pl.ds(cs, 16)]) * 5.0
                g_h = _logf16((y2 - y1) * rcp_v[3, pl.ds(cs, 16)]) * 5.0
                conf = jnp.where(ev >= 32, (lb + 1.0).astype(jnp.int32), 0)
                loc_v[i, 0, pl.ds(cs, 16)] = g_cx
                loc_v[i, 1, pl.ds(cs, 16)] = g_cy
                loc_v[i, 2, pl.ds(cs, 16)] = g_w
                loc_v[i, 3, pl.ds(cs, 16)] = g_h
                conf_v[i, pl.ds(cs, 16)] = conf
            return carry

        lax.fori_loop(0, n, one_image, 0)
        pltpu.sync_copy(loc_v.at[pl.ds(0, n)],
                        loc_hbm.at[pl.ds(ibm, n), :, pl.ds(base, 128)])
        pltpu.sync_copy(conf_v.at[pl.ds(0, n)],
                        conf_hbm.at[pl.ds(ibm, n), pl.ds(base, 128)])

    process(wid, 0, B)
    process(NW + wid, 0, B)
    process(2 * NW + wid // 4, (wid % 4) * 8, 8)


_SC_ENCODE_CACHE = []


def _sc_encode(*args):
    if not _SC_ENCODE_CACHE:
        _SC_ENCODE_CACHE.append(_make_sc_encode())
    return _SC_ENCODE_CACHE[0](*args)


def _make_sc_encode():
    return functools.partial(
        pl.kernel,
        out_type=[
            jax.ShapeDtypeStruct((B, 4, PPAD), jnp.float32),
            jax.ShapeDtypeStruct((B, PPAD), jnp.int32),
        ],
        mesh=plsc.VectorSubcoreMesh(core_axis_name="c", subcore_axis_name="s",
                                    num_cores=2, num_subcores=16),
        compiler_params=pltpu.CompilerParams(needs_layout_passes=False),
        scratch_types=[
            pltpu.VMEM((4, 128), jnp.float32),
            pltpu.VMEM((4, 128), jnp.float32),
            pltpu.VMEM((B * 8 * 32,), jnp.float32),
            pltpu.VMEM((B, 1, 128), jnp.float32),
            pltpu.VMEM((B, 4, 128), jnp.float32),
            pltpu.VMEM((B, 128), jnp.int32),
        ],
    )(_sc_encode_body)


def kernel(loc_data, conf_data, priors, targets):
    del loc_data, conf_data
    pri = priors[:P, :]
    pad_pri = jnp.broadcast_to(jnp.array([-50.0, -50.0, 1.0, 1.0],
                                         jnp.float32), (PPAD - P, 4))
    p4 = jnp.concatenate([pri, pad_pri], axis=0).T

    pad_box = jnp.array([-9.0, -9.0, -8.0, -8.0, 0.0], jnp.float32)
    tgt24 = jnp.concatenate(
        [targets, jnp.broadcast_to(pad_box, (B, TROWS - O, 5))], axis=1)
    ttm = jnp.pad(tgt24, ((0, 0), (0, 0), (0, 128 - 5)))
    tgt32 = jnp.concatenate(
        [targets, jnp.broadcast_to(pad_box, (B, 32 - O, 5))], axis=1)
    t_sc = jnp.pad(jnp.swapaxes(tgt32, 1, 2),
                   ((0, 0), (0, 3), (0, 0))).reshape(-1)

    [enc] = pl.pallas_call(
        _match_body,
        grid=(B,),
        in_specs=[
            pl.BlockSpec((4, PPAD), lambda i: (0, 0)),
            pl.BlockSpec((1, TROWS, 128), lambda i: (i, 0, 0)),
        ],
        out_specs=[
            pl.BlockSpec((1, 1, PPAD), lambda i: (i, 0, 0)),
        ],
        out_shape=[
            jax.ShapeDtypeStruct((B, 1, PPAD), jnp.float32),
        ],
    )(p4, ttm)

    loc_p, conf_p = _sc_encode(p4, t_sc, enc)
    return jnp.swapaxes(loc_p, 1, 2)[:, :P, :], conf_p[:, :P]


# speedup vs baseline: 1.2958x; 1.0549x over previous
import functools

import jax
import jax.numpy as jnp
from jax import lax
from jax.experimental import pallas as pl
from jax.experimental.pallas import tpu as pltpu
from jax.experimental.pallas import tpu_sc as plsc

B, P, C, O = 32, 8732, 21, 20
PPAD = 9216
TROWS = 24
THRESH = 0.5
NW = 32
NU = PPAD // 128
BH = 16
QH = BH // 4


def _match_body(p4_ref, ttm_ref, bti_ref):
    p4 = p4_ref[...]
    pcx, pcy = p4[0:1, :], p4[1:2, :]
    pw, ph = p4[2:3, :], p4[3:4, :]
    px1 = pcx - pw / 2.0
    py1 = pcy - ph / 2.0
    px2 = pcx + pw / 2.0
    py2 = pcy + ph / 2.0

    ttm = ttm_ref[0]
    tx1 = ttm[:, 0:1]
    ty1 = ttm[:, 1:2]
    tx2 = ttm[:, 2:3]
    ty2 = ttm[:, 3:4]

    iw = jnp.clip(jnp.minimum(tx2, px2) - jnp.maximum(tx1, px1), 0.0, None)
    ih = jnp.clip(jnp.minimum(ty2, py2) - jnp.maximum(ty1, py1), 0.0, None)
    inter = iw * ih
    area_a = (tx2 - tx1) * (ty2 - ty1)
    area_b = (px2 - px1) * (py2 - py1)
    union = area_a + area_b - inter
    ov = inter / union

    rows = lax.broadcasted_iota(jnp.int32, (TROWS, PPAD), 0)
    lanes = lax.broadcasted_iota(jnp.int32, (TROWS, PPAD), 1)

    bto = jnp.max(ov, axis=0, keepdims=True)
    bti = jnp.min(jnp.where(ov == bto, rows, TROWS),
                  axis=0, keepdims=True)
    bpv = jnp.max(ov, axis=1, keepdims=True)
    bpi = jnp.min(jnp.where(ov == bpv, lanes, PPAD),
                  axis=1, keepdims=True)

    match = (lanes == bpi) & (rows < O)
    j_forced = jnp.max(jnp.where(match, rows, -1), axis=0, keepdims=True)
    bti2 = jnp.where(j_forced >= 0, j_forced, bti)
    maskok = (bto >= THRESH) | (j_forced >= 0)
    bti_ref[0] = (bti2 + 32 * maskok.astype(jnp.int32)).astype(jnp.float32)


def _logf16(x):
    bits = lax.bitcast_convert_type(x, jnp.int32)
    e = ((bits >> 23) & 0xFF) - 126
    m = lax.bitcast_convert_type((bits & 0x007FFFFF) | 0x3F000000,
                                 jnp.float32)
    c = m < 0.7071067811865476
    m = jnp.where(c, m + m, m)
    ef = (e - jnp.where(c, 1, 0)).astype(jnp.float32)
    z = m - 1.0
    zz = z * z
    poly = jnp.full((16,), -1.0 / 6.0, jnp.float32)
    for k in (0.2, -0.25, 1.0 / 3.0):
        poly = poly * z + jnp.float32(k)
    y = z * zz * poly
    y = y + ef * jnp.float32(-2.12194440e-4)
    y = y - 0.5 * zz
    return z + y + ef * jnp.float32(0.693359375)


def _sc_encode_body(p4_hbm, t_hbm, enc_hbm, loc_hbm, conf_hbm,
                    pri_v, rcp_v, t_v, enc_v, loc_v, conf_v):
    wid = lax.axis_index("s") * 2 + lax.axis_index("c")
    pltpu.sync_copy(t_hbm, t_v)
    lane = lax.broadcasted_iota(jnp.int32, (16,), 0)

    def process(unit, ib, n):
        base = pl.multiple_of(unit * 128, 128)
        ibm = ib
        pltpu.sync_copy(p4_hbm.at[:, pl.ds(base, 128)], pri_v)
        pltpu.sync_copy(enc_hbm.at[pl.ds(ibm, n), :, pl.ds(base, 128)],
                        enc_v.at[pl.ds(0, n)])
        for ch in range(8):
            cs = ch * 16
            rcp_v[0, pl.ds(cs, 16)] = 1.0 / (0.1 * pri_v[2, pl.ds(cs, 16)])
            rcp_v[1, pl.ds(cs, 16)] = 1.0 / (0.1 * pri_v[3, pl.ds(cs, 16)])
            rcp_v[2, pl.ds(cs, 16)] = 1.0 / pri_v[2, pl.ds(cs, 16)]
            rcp_v[3, pl.ds(cs, 16)] = 1.0 / pri_v[3, pl.ds(cs, 16)]

        def one_image(i, carry):
            iv = jnp.full((16,), i, jnp.int32)
            tbase = (ibm + i) * 256
            for ch in range(8):
                cs = ch * 16
                col = lane + cs
                ev = enc_v[i, 0, pl.ds(cs, 16)].astype(jnp.int32)
                idx = ev & 31
                ti = tbase + idx
                x1 = plsc.load_gather(t_v, [ti])
                y1 = plsc.load_gather(t_v, [ti + 32])
                x2 = plsc.load_gather(t_v, [ti + 64])
                y2 = plsc.load_gather(t_v, [ti + 96])
                lb = plsc.load_gather(t_v, [ti + 128])
                pcx = pri_v[0, pl.ds(cs, 16)]
                pcy = pri_v[1, pl.ds(cs, 16)]
                g_cx = ((x1 + x2) * 0.5 - pcx) * rcp_v[0, pl.ds(cs, 16)]
                g_cy = ((y1 + y2) * 0.5 - pcy) * rcp_v[1, pl.ds(cs, 16)]
                g_w = _logf16((x2 - x1) * rcp_v[2, pl.ds(cs, 16)]) * 5.0
                g_h = _logf16((y2 - y1) * rcp_v[3, pl.ds(cs, 16)]) * 5.0
                conf = jnp.where(ev >= 32, (lb + 1.0).astype(jnp.int32), 0)
                loc_v[i, 0, pl.ds(cs, 16)] = g_cx
                loc_v[i, 1, pl.ds(cs, 16)] = g_cy
                loc_v[i, 2, pl.ds(cs, 16)] = g_w
                loc_v[i, 3, pl.ds(cs, 16)] = g_h
                conf_v[i, 0, pl.ds(cs, 16)] = conf
            return carry

        lax.fori_loop(0, n, one_image, 0)
        pltpu.sync_copy(loc_v.at[pl.ds(0, n)],
                        loc_hbm.at[pl.ds(ibm, n), :, pl.ds(base, 128)])
        pltpu.sync_copy(conf_v.at[pl.ds(0, n)],
                        conf_hbm.at[pl.ds(ibm, n), :, pl.ds(base, 128)])

    process(wid, 0, BH)
    process(NW + wid, 0, BH)
    process(2 * NW + wid // 4, (wid % 4) * QH, QH)


_SC_ENCODE_CACHE = []


def _sc_encode(*args):
    if not _SC_ENCODE_CACHE:
        _SC_ENCODE_CACHE.append(_make_sc_encode())
    return _SC_ENCODE_CACHE[0](*args)


def _make_sc_encode():
    return functools.partial(
        pl.kernel,
        out_type=[
            jax.ShapeDtypeStruct((BH, 4, PPAD), jnp.float32),
            jax.ShapeDtypeStruct((BH, 1, PPAD), jnp.int32),
        ],
        mesh=plsc.VectorSubcoreMesh(core_axis_name="c", subcore_axis_name="s",
                                    num_cores=2, num_subcores=16),
        compiler_params=pltpu.CompilerParams(needs_layout_passes=False),
        scratch_types=[
            pltpu.VMEM((4, 128), jnp.float32),
            pltpu.VMEM((4, 128), jnp.float32),
            pltpu.VMEM((BH * 8 * 32,), jnp.float32),
            pltpu.VMEM((BH, 1, 128), jnp.float32),
            pltpu.VMEM((BH, 4, 128), jnp.float32),
            pltpu.VMEM((BH, 1, 128), jnp.int32),
        ],
    )(_sc_encode_body)


def kernel(loc_data, conf_data, priors, targets):
    del loc_data, conf_data
    pri = priors[:P, :]
    pad_pri = jnp.broadcast_to(jnp.array([-50.0, -50.0, 1.0, 1.0],
                                         jnp.float32), (PPAD - P, 4))
    p4 = jnp.concatenate([pri, pad_pri], axis=0).T

    pad_box = jnp.array([-9.0, -9.0, -8.0, -8.0, 0.0], jnp.float32)
    tgt24 = jnp.concatenate(
        [targets, jnp.broadcast_to(pad_box, (B, TROWS - O, 5))], axis=1)
    ttm = jnp.pad(tgt24, ((0, 0), (0, 0), (0, 128 - 5)))
    tgt32 = jnp.concatenate(
        [targets, jnp.broadcast_to(pad_box, (B, 32 - O, 5))], axis=1)
    t_sc = jnp.pad(jnp.swapaxes(tgt32, 1, 2),
                   ((0, 0), (0, 3), (0, 0))).reshape(-1)

    locs, confs = [], []
    for h in range(2):
        [enc] = pl.pallas_call(
            _match_body,
            grid=(BH,),
            in_specs=[
                pl.BlockSpec((4, PPAD), lambda i: (0, 0)),
                pl.BlockSpec((1, TROWS, 128), lambda i: (i, 0, 0)),
            ],
            out_specs=[
                pl.BlockSpec((1, 1, PPAD), lambda i: (i, 0, 0)),
            ],
            out_shape=[
                jax.ShapeDtypeStruct((BH, 1, PPAD), jnp.float32),
            ],
        )(p4, ttm[h * BH:(h + 1) * BH])
        loc_h, conf_h = _sc_encode(
            p4, t_sc[h * BH * 256:(h + 1) * BH * 256], enc)
        locs.append(jnp.swapaxes(loc_h, 1, 2)[:, :P, :])
        confs.append(conf_h[:, 0, :P])
    return jnp.concatenate(locs), jnp.concatenate(confs)


# speedup vs baseline: 1.3514x; 1.0429x over previous
import functools

import jax
import jax.numpy as jnp
from jax import lax
from jax.experimental import pallas as pl
from jax.experimental.pallas import tpu as pltpu
from jax.experimental.pallas import tpu_sc as plsc

B, P, C, O = 32, 8732, 21, 20
PPAD = 9216
TROWS = 24
THRESH = 0.5
NW = 32
NU = PPAD // 128
BH = 16
QH = BH // 4


def _match_body(p4_ref, ttm_ref, bti_ref):
    p4 = p4_ref[...]
    pcx, pcy = p4[0:1, :], p4[1:2, :]
    pw, ph = p4[2:3, :], p4[3:4, :]
    px1 = pcx - pw / 2.0
    py1 = pcy - ph / 2.0
    px2 = pcx + pw / 2.0
    py2 = pcy + ph / 2.0

    ttm = ttm_ref[0]
    tx1 = ttm[:, 0:1]
    ty1 = ttm[:, 1:2]
    tx2 = ttm[:, 2:3]
    ty2 = ttm[:, 3:4]

    iw = jnp.clip(jnp.minimum(tx2, px2) - jnp.maximum(tx1, px1), 0.0, None)
    ih = jnp.clip(jnp.minimum(ty2, py2) - jnp.maximum(ty1, py1), 0.0, None)
    inter = iw * ih
    area_a = (tx2 - tx1) * (ty2 - ty1)
    area_b = (px2 - px1) * (py2 - py1)
    union = area_a + area_b - inter
    ov = inter / union

    rows = lax.broadcasted_iota(jnp.int32, (TROWS, PPAD), 0)
    lanes = lax.broadcasted_iota(jnp.int32, (TROWS, PPAD), 1)

    bto = jnp.max(ov, axis=0, keepdims=True)
    bti = jnp.min(jnp.where(ov == bto, rows, TROWS),
                  axis=0, keepdims=True)
    bpv = jnp.max(ov, axis=1, keepdims=True)
    bpi = jnp.min(jnp.where(ov == bpv, lanes, PPAD),
                  axis=1, keepdims=True)

    match = (lanes == bpi) & (rows < O)
    j_forced = jnp.max(jnp.where(match, rows, -1), axis=0, keepdims=True)
    bti2 = jnp.where(j_forced >= 0, j_forced, bti)
    maskok = (bto >= THRESH) | (j_forced >= 0)
    bti_ref[0] = (bti2 + 32 * maskok.astype(jnp.int32)).astype(jnp.float32)


def _logf16(x):
    bits = lax.bitcast_convert_type(x, jnp.int32)
    e = ((bits >> 23) & 0xFF) - 126
    m = lax.bitcast_convert_type((bits & 0x007FFFFF) | 0x3F000000,
                                 jnp.float32)
    c = m < 0.7071067811865476
    m = jnp.where(c, m + m, m)
    ef = (e - jnp.where(c, 1, 0)).astype(jnp.float32)
    z = m - 1.0
    zz = z * z
    poly = jnp.full((16,), -1.0 / 6.0, jnp.float32)
    for k in (0.2, -0.25, 1.0 / 3.0):
        poly = poly * z + jnp.float32(k)
    y = z * zz * poly
    y = y + ef * jnp.float32(-2.12194440e-4)
    y = y - 0.5 * zz
    return z + y + ef * jnp.float32(0.693359375)


def _sc_encode_body(p4_hbm, t_hbm, enc_hbm, loc_hbm, conf_hbm,
                    pri0, pri1, rcp_v, t_v, enc0, enc1,
                    loc0, loc1, conf0, conf1,
                    s_t, s_p0, s_e0, s_p1, s_e1, s_l0, s_l1, s_c0, s_c1):
    wid = lax.axis_index("s") * 2 + lax.axis_index("c")
    lane = lax.broadcasted_iota(jnp.int32, (16,), 0)
    units = [wid, NW + wid, 2 * NW + wid // 4]
    ibs = [0, 0, (wid % 4) * QH]
    ns = [BH, BH, QH]

    def issue_in(s, pri_v, enc_v, s_p, s_e):
        base = pl.multiple_of(units[s] * 128, 128)
        hp = pltpu.async_copy(p4_hbm.at[:, pl.ds(base, 128)], pri_v, s_p)
        he = pltpu.async_copy(
            enc_hbm.at[pl.ds(ibs[s], ns[s]), :, pl.ds(base, 128)],
            enc_v.at[pl.ds(0, ns[s])], s_e)
        return hp, he

    def compute(s, pri_v, enc_v, loc_v, conf_v):
        ib, n = ibs[s], ns[s]
        for ch in range(8):
            cs = ch * 16
            rcp_v[0, pl.ds(cs, 16)] = 1.0 / (0.1 * pri_v[2, pl.ds(cs, 16)])
            rcp_v[1, pl.ds(cs, 16)] = 1.0 / (0.1 * pri_v[3, pl.ds(cs, 16)])
            rcp_v[2, pl.ds(cs, 16)] = 1.0 / pri_v[2, pl.ds(cs, 16)]
            rcp_v[3, pl.ds(cs, 16)] = 1.0 / pri_v[3, pl.ds(cs, 16)]

        def one_image(i, carry):
            tbase = (ib + i) * 256
            for ch in range(8):
                cs = ch * 16
                ev = enc_v[i, 0, pl.ds(cs, 16)].astype(jnp.int32)
                idx = ev & 31
                ti = tbase + idx
                x1 = plsc.load_gather(t_v, [ti])
                y1 = plsc.load_gather(t_v, [ti + 32])
                x2 = plsc.load_gather(t_v, [ti + 64])
                y2 = plsc.load_gather(t_v, [ti + 96])
                lb = plsc.load_gather(t_v, [ti + 128])
                pcx = pri_v[0, pl.ds(cs, 16)]
                pcy = pri_v[1, pl.ds(cs, 16)]
                g_cx = ((x1 + x2) * 0.5 - pcx) * rcp_v[0, pl.ds(cs, 16)]
                g_cy = ((y1 + y2) * 0.5 - pcy) * rcp_v[1, pl.ds(cs, 16)]
                g_w = _logf16((x2 - x1) * rcp_v[2, pl.ds(cs, 16)]) * 5.0
                g_h = _logf16((y2 - y1) * rcp_v[3, pl.ds(cs, 16)]) * 5.0
                conf = jnp.where(ev >= 32, (lb + 1.0).astype(jnp.int32), 0)
                loc_v[i, 0, pl.ds(cs, 16)] = g_cx
                loc_v[i, 1, pl.ds(cs, 16)] = g_cy
                loc_v[i, 2, pl.ds(cs, 16)] = g_w
                loc_v[i, 3, pl.ds(cs, 16)] = g_h
                conf_v[i, 0, pl.ds(cs, 16)] = conf
            return carry

        lax.fori_loop(0, n, one_image, 0)

    def issue_out(s, loc_v, conf_v, s_l, s_c):
        base = pl.multiple_of(units[s] * 128, 128)
        ib, n = ibs[s], ns[s]
        hl = pltpu.async_copy(loc_v.at[pl.ds(0, n)],
                              loc_hbm.at[pl.ds(ib, n), :, pl.ds(base, 128)],
                              s_l)
        hc = pltpu.async_copy(conf_v.at[pl.ds(0, n)],
                              conf_hbm.at[pl.ds(ib, n), :, pl.ds(base, 128)],
                              s_c)
        return hl, hc

    ht = pltpu.async_copy(t_hbm, t_v, s_t)
    hp0, he0 = issue_in(0, pri0, enc0, s_p0, s_e0)
    hp1, he1 = issue_in(1, pri1, enc1, s_p1, s_e1)
    ht.wait()
    hp0.wait()
    he0.wait()
    compute(0, pri0, enc0, loc0, conf0)
    hl0, hc0 = issue_out(0, loc0, conf0, s_l0, s_c0)
    hp2, he2 = issue_in(2, pri0, enc0, s_p0, s_e0)
    hp1.wait()
    he1.wait()
    compute(1, pri1, enc1, loc1, conf1)
    hl1, hc1 = issue_out(1, loc1, conf1, s_l1, s_c1)
    hp2.wait()
    he2.wait()
    hl0.wait()
    hc0.wait()
    compute(2, pri0, enc0, loc0, conf0)
    hl2, hc2 = issue_out(2, loc0, conf0, s_l0, s_c0)
    hl1.wait()
    hc1.wait()
    hl2.wait()
    hc2.wait()


_SC_ENCODE_CACHE = []


def _sc_encode(*args):
    if not _SC_ENCODE_CACHE:
        _SC_ENCODE_CACHE.append(_make_sc_encode())
    return _SC_ENCODE_CACHE[0](*args)


def _make_sc_encode():
    return functools.partial(
        pl.kernel,
        out_type=[
            jax.ShapeDtypeStruct((BH, 4, PPAD), jnp.float32),
            jax.ShapeDtypeStruct((BH, 1, PPAD), jnp.int32),
        ],
        mesh=plsc.VectorSubcoreMesh(core_axis_name="c", subcore_axis_name="s",
                                    num_cores=2, num_subcores=16),
        compiler_params=pltpu.CompilerParams(needs_layout_passes=False),
        scratch_types=[
            pltpu.VMEM((4, 128), jnp.float32),
            pltpu.VMEM((4, 128), jnp.float32),
            pltpu.VMEM((4, 128), jnp.float32),
            pltpu.VMEM((BH * 8 * 32,), jnp.float32),
            pltpu.VMEM((BH, 1, 128), jnp.float32),
            pltpu.VMEM((BH, 1, 128), jnp.float32),
            pltpu.VMEM((BH, 4, 128), jnp.float32),
            pltpu.VMEM((BH, 4, 128), jnp.float32),
            pltpu.VMEM((BH, 1, 128), jnp.int32),
            pltpu.VMEM((BH, 1, 128), jnp.int32),
        ] + [pltpu.SemaphoreType.DMA] * 9,
    )(_sc_encode_body)


def kernel(loc_data, conf_data, priors, targets):
    del loc_data, conf_data
    pri = priors[:P, :]
    pad_pri = jnp.broadcast_to(jnp.array([-50.0, -50.0, 1.0, 1.0],
                                         jnp.float32), (PPAD - P, 4))
    p4 = jnp.concatenate([pri, pad_pri], axis=0).T

    pad_box = jnp.array([-9.0, -9.0, -8.0, -8.0, 0.0], jnp.float32)
    tgt24 = jnp.concatenate(
        [targets, jnp.broadcast_to(pad_box, (B, TROWS - O, 5))], axis=1)
    ttm = jnp.pad(tgt24, ((0, 0), (0, 0), (0, 128 - 5)))
    tgt32 = jnp.concatenate(
        [targets, jnp.broadcast_to(pad_box, (B, 32 - O, 5))], axis=1)
    t_sc = jnp.pad(jnp.swapaxes(tgt32, 1, 2),
                   ((0, 0), (0, 3), (0, 0))).reshape(-1)

    locs, confs = [], []
    for h in range(2):
        [enc] = pl.pallas_call(
            _match_body,
            grid=(BH,),
            in_specs=[
                pl.BlockSpec((4, PPAD), lambda i: (0, 0)),
                pl.BlockSpec((1, TROWS, 128), lambda i: (i, 0, 0)),
            ],
            out_specs=[
                pl.BlockSpec((1, 1, PPAD), lambda i: (i, 0, 0)),
            ],
            out_shape=[
                jax.ShapeDtypeStruct((BH, 1, PPAD), jnp.float32),
            ],
        )(p4, ttm[h * BH:(h + 1) * BH])
        loc_h, conf_h = _sc_encode(
            p4, t_sc[h * BH * 256:(h + 1) * BH * 256], enc)
        locs.append(jnp.swapaxes(loc_h, 1, 2)[:, :P, :])
        confs.append(conf_h[:, 0, :P])
    return jnp.concatenate(locs), jnp.concatenate(confs)


# speedup vs baseline: 1.4086x; 1.0424x over previous
import functools

import jax
import jax.numpy as jnp
from jax import lax
from jax.experimental import pallas as pl
from jax.experimental.pallas import tpu as pltpu
from jax.experimental.pallas import tpu_sc as plsc

B, P, C, O = 32, 8732, 21, 20
PPAD = 9216
TROWS = 24
THRESH = 0.5
NW = 32
NU = PPAD // 128
BH = 8
QH = BH // 4


def _match_body(p4_ref, ttm_ref, bti_ref):
    p4 = p4_ref[...]
    pcx, pcy = p4[0:1, :], p4[1:2, :]
    pw, ph = p4[2:3, :], p4[3:4, :]
    px1 = pcx - pw / 2.0
    py1 = pcy - ph / 2.0
    px2 = pcx + pw / 2.0
    py2 = pcy + ph / 2.0

    ttm = ttm_ref[0]
    tx1 = ttm[:, 0:1]
    ty1 = ttm[:, 1:2]
    tx2 = ttm[:, 2:3]
    ty2 = ttm[:, 3:4]

    iw = jnp.clip(jnp.minimum(tx2, px2) - jnp.maximum(tx1, px1), 0.0, None)
    ih = jnp.clip(jnp.minimum(ty2, py2) - jnp.maximum(ty1, py1), 0.0, None)
    inter = iw * ih
    area_a = (tx2 - tx1) * (ty2 - ty1)
    area_b = (px2 - px1) * (py2 - py1)
    union = area_a + area_b - inter
    ov = inter / union

    rows = lax.broadcasted_iota(jnp.int32, (TROWS, PPAD), 0)
    lanes = lax.broadcasted_iota(jnp.int32, (TROWS, PPAD), 1)

    bto = jnp.max(ov, axis=0, keepdims=True)
    bti = jnp.min(jnp.where(ov == bto, rows, TROWS),
                  axis=0, keepdims=True)
    bpv = jnp.max(ov, axis=1, keepdims=True)
    bpi = jnp.min(jnp.where(ov == bpv, lanes, PPAD),
                  axis=1, keepdims=True)

    match = (lanes == bpi) & (rows < O)
    j_forced = jnp.max(jnp.where(match, rows, -1), axis=0, keepdims=True)
    bti2 = jnp.where(j_forced >= 0, j_forced, bti)
    maskok = (bto >= THRESH) | (j_forced >= 0)
    bti_ref[0] = (bti2 + 32 * maskok.astype(jnp.int32)).astype(jnp.float32)


def _logf16(x):
    bits = lax.bitcast_convert_type(x, jnp.int32)
    e = ((bits >> 23) & 0xFF) - 126
    m = lax.bitcast_convert_type((bits & 0x007FFFFF) | 0x3F000000,
                                 jnp.float32)
    c = m < 0.7071067811865476
    m = jnp.where(c, m + m, m)
    ef = (e - jnp.where(c, 1, 0)).astype(jnp.float32)
    z = m - 1.0
    zz = z * z
    poly = jnp.full((16,), -1.0 / 6.0, jnp.float32)
    for k in (0.2, -0.25, 1.0 / 3.0):
        poly = poly * z + jnp.float32(k)
    y = z * zz * poly
    y = y + ef * jnp.float32(-2.12194440e-4)
    y = y - 0.5 * zz
    return z + y + ef * jnp.float32(0.693359375)


def _sc_encode_body(p4_hbm, t_hbm, enc_hbm, loc_hbm, conf_hbm,
                    pri0, pri1, rcp_v, t_v, enc0, enc1,
                    loc0, loc1, conf0, conf1,
                    s_t, s_p0, s_e0, s_p1, s_e1, s_l0, s_l1, s_c0, s_c1):
    wid = lax.axis_index("s") * 2 + lax.axis_index("c")
    lane = lax.broadcasted_iota(jnp.int32, (16,), 0)
    units = [wid, NW + wid, 2 * NW + wid // 4]
    ibs = [0, 0, (wid % 4) * QH]
    ns = [BH, BH, QH]

    def issue_in(s, pri_v, enc_v, s_p, s_e):
        base = pl.multiple_of(units[s] * 128, 128)
        hp = pltpu.async_copy(p4_hbm.at[:, pl.ds(base, 128)], pri_v, s_p)
        he = pltpu.async_copy(
            enc_hbm.at[pl.ds(ibs[s], ns[s]), :, pl.ds(base, 128)],
            enc_v.at[pl.ds(0, ns[s])], s_e)
        return hp, he

    def compute(s, pri_v, enc_v, loc_v, conf_v):
        ib, n = ibs[s], ns[s]
        for ch in range(8):
            cs = ch * 16
            rcp_v[0, pl.ds(cs, 16)] = 1.0 / (0.1 * pri_v[2, pl.ds(cs, 16)])
            rcp_v[1, pl.ds(cs, 16)] = 1.0 / (0.1 * pri_v[3, pl.ds(cs, 16)])
            rcp_v[2, pl.ds(cs, 16)] = 1.0 / pri_v[2, pl.ds(cs, 16)]
            rcp_v[3, pl.ds(cs, 16)] = 1.0 / pri_v[3, pl.ds(cs, 16)]

        def one_image(i, carry):
            tbase = (ib + i) * 256
            for ch in range(8):
                cs = ch * 16
                ev = enc_v[i, 0, pl.ds(cs, 16)].astype(jnp.int32)
                idx = ev & 31
                ti = tbase + idx
                x1 = plsc.load_gather(t_v, [ti])
                y1 = plsc.load_gather(t_v, [ti + 32])
                x2 = plsc.load_gather(t_v, [ti + 64])
                y2 = plsc.load_gather(t_v, [ti + 96])
                lb = plsc.load_gather(t_v, [ti + 128])
                pcx = pri_v[0, pl.ds(cs, 16)]
                pcy = pri_v[1, pl.ds(cs, 16)]
                g_cx = ((x1 + x2) * 0.5 - pcx) * rcp_v[0, pl.ds(cs, 16)]
                g_cy = ((y1 + y2) * 0.5 - pcy) * rcp_v[1, pl.ds(cs, 16)]
                g_w = _logf16((x2 - x1) * rcp_v[2, pl.ds(cs, 16)]) * 5.0
                g_h = _logf16((y2 - y1) * rcp_v[3, pl.ds(cs, 16)]) * 5.0
                conf = jnp.where(ev >= 32, (lb + 1.0).astype(jnp.int32), 0)
                loc_v[i, 0, pl.ds(cs, 16)] = g_cx
                loc_v[i, 1, pl.ds(cs, 16)] = g_cy
                loc_v[i, 2, pl.ds(cs, 16)] = g_w
                loc_v[i, 3, pl.ds(cs, 16)] = g_h
                conf_v[i, 0, pl.ds(cs, 16)] = conf
            return carry

        lax.fori_loop(0, n, one_image, 0)

    def issue_out(s, loc_v, conf_v, s_l, s_c):
        base = pl.multiple_of(units[s] * 128, 128)
        ib, n = ibs[s], ns[s]
        hl = pltpu.async_copy(loc_v.at[pl.ds(0, n)],
                              loc_hbm.at[pl.ds(ib, n), :, pl.ds(base, 128)],
                              s_l)
        hc = pltpu.async_copy(conf_v.at[pl.ds(0, n)],
                              conf_hbm.at[pl.ds(ib, n), :, pl.ds(base, 128)],
                              s_c)
        return hl, hc

    ht = pltpu.async_copy(t_hbm, t_v, s_t)
    hp0, he0 = issue_in(0, pri0, enc0, s_p0, s_e0)
    hp1, he1 = issue_in(1, pri1, enc1, s_p1, s_e1)
    ht.wait()
    hp0.wait()
    he0.wait()
    compute(0, pri0, enc0, loc0, conf0)
    hl0, hc0 = issue_out(0, loc0, conf0, s_l0, s_c0)
    hp2, he2 = issue_in(2, pri0, enc0, s_p0, s_e0)
    hp1.wait()
    he1.wait()
    compute(1, pri1, enc1, loc1, conf1)
    hl1, hc1 = issue_out(1, loc1, conf1, s_l1, s_c1)
    hp2.wait()
    he2.wait()
    hl0.wait()
    hc0.wait()
    compute(2, pri0, enc0, loc0, conf0)
    hl2, hc2 = issue_out(2, loc0, conf0, s_l0, s_c0)
    hl1.wait()
    hc1.wait()
    hl2.wait()
    hc2.wait()


_SC_ENCODE_CACHE = []


def _sc_encode(*args):
    if not _SC_ENCODE_CACHE:
        _SC_ENCODE_CACHE.append(_make_sc_encode())
    return _SC_ENCODE_CACHE[0](*args)


def _make_sc_encode():
    return functools.partial(
        pl.kernel,
        out_type=[
            jax.ShapeDtypeStruct((BH, 4, PPAD), jnp.float32),
            jax.ShapeDtypeStruct((BH, 1, PPAD), jnp.int32),
        ],
        mesh=plsc.VectorSubcoreMesh(core_axis_name="c", subcore_axis_name="s",
                                    num_cores=2, num_subcores=16),
        compiler_params=pltpu.CompilerParams(needs_layout_passes=False),
        scratch_types=[
            pltpu.VMEM((4, 128), jnp.float32),
            pltpu.VMEM((4, 128), jnp.float32),
            pltpu.VMEM((4, 128), jnp.float32),
            pltpu.VMEM((BH * 8 * 32,), jnp.float32),
            pltpu.VMEM((BH, 1, 128), jnp.float32),
            pltpu.VMEM((BH, 1, 128), jnp.float32),
            pltpu.VMEM((BH, 4, 128), jnp.float32),
            pltpu.VMEM((BH, 4, 128), jnp.float32),
            pltpu.VMEM((BH, 1, 128), jnp.int32),
            pltpu.VMEM((BH, 1, 128), jnp.int32),
        ] + [pltpu.SemaphoreType.DMA] * 9,
    )(_sc_encode_body)


def kernel(loc_data, conf_data, priors, targets):
    del loc_data, conf_data
    pri = priors[:P, :]
    pad_pri = jnp.broadcast_to(jnp.array([-50.0, -50.0, 1.0, 1.0],
                                         jnp.float32), (PPAD - P, 4))
    p4 = jnp.concatenate([pri, pad_pri], axis=0).T

    pad_box = jnp.array([-9.0, -9.0, -8.0, -8.0, 0.0], jnp.float32)
    tgt24 = jnp.concatenate(
        [targets, jnp.broadcast_to(pad_box, (B, TROWS - O, 5))], axis=1)
    ttm = jnp.pad(tgt24, ((0, 0), (0, 0), (0, 128 - 5)))
    tgt32 = jnp.concatenate(
        [targets, jnp.broadcast_to(pad_box, (B, 32 - O, 5))], axis=1)
    t_sc = jnp.pad(jnp.swapaxes(tgt32, 1, 2),
                   ((0, 0), (0, 3), (0, 0))).reshape(-1)

    locs, confs = [], []
    for h in range(B // BH):
        [enc] = pl.pallas_call(
            _match_body,
            grid=(BH,),
            in_specs=[
                pl.BlockSpec((4, PPAD), lambda i: (0, 0)),
                pl.BlockSpec((1, TROWS, 128), lambda i: (i, 0, 0)),
            ],
            out_specs=[
                pl.BlockSpec((1, 1, PPAD), lambda i: (i, 0, 0)),
            ],
            out_shape=[
                jax.ShapeDtypeStruct((BH, 1, PPAD), jnp.float32),
            ],
        )(p4, ttm[h * BH:(h + 1) * BH])
        loc_h, conf_h = _sc_encode(
            p4, t_sc[h * BH * 256:(h + 1) * BH * 256], enc)
        locs.append(jnp.swapaxes(loc_h, 1, 2)[:, :P, :])
        confs.append(conf_h[:, 0, :P])
    return jnp.concatenate(locs), jnp.concatenate(confs)
